# Initial kernel scaffold; baseline (speedup 1.0000x reference)
import jax
import jax.numpy as jnp
from jax.experimental import pallas as pl


def kernel(x, edge_index, edge_attr, batch, params):
    raise NotImplementedError("write your pallas kernel here")


# speedup vs baseline: 1.0331x; 1.0331x over previous
import functools

import jax
import jax.numpy as jnp
from jax.experimental import pallas as pl
from jax.experimental.pallas import tpu as pltpu

_N = 10000
_G = 64
_C = 10
_RB = 1000
_NB = _N // _RB

_BN_SCALE = 1.0 / (1.0 + 1e-5) ** 0.5


def _layer_body(z_ref, a_ref, w1_ref, b1_ref, w2_ref, b2_ref, s_ref, t_ref, o_ref):
    h = z_ref[...] + a_ref[...]
    u = jnp.maximum(jnp.dot(h, w1_ref[...], preferred_element_type=jnp.float32)
                    + b1_ref[...], 0.0)
    y = jnp.dot(u, w2_ref[...], preferred_element_type=jnp.float32) + b2_ref[...]
    o_ref[...] = jnp.maximum(y, 0.0) * s_ref[...] + t_ref[...]


def _tc_layer(z, a, w1, b1, w2, b2, g, bb):
    din, dout = w1.shape
    s = (g * _BN_SCALE).reshape(1, dout)
    t = bb.reshape(1, dout)
    return pl.pallas_call(
        _layer_body,
        grid=(_NB,),
        in_specs=[
            pl.BlockSpec((_RB, din), lambda i: (i, 0)),
            pl.BlockSpec((_RB, din), lambda i: (i, 0)),
            pl.BlockSpec((din, dout), lambda i: (0, 0)),
            pl.BlockSpec((1, dout), lambda i: (0, 0)),
            pl.BlockSpec((dout, dout), lambda i: (0, 0)),
            pl.BlockSpec((1, dout), lambda i: (0, 0)),
            pl.BlockSpec((1, dout), lambda i: (0, 0)),
            pl.BlockSpec((1, dout), lambda i: (0, 0)),
        ],
        out_specs=pl.BlockSpec((_RB, dout), lambda i: (i, 0)),
        out_shape=jax.ShapeDtypeStruct((_N, dout), jnp.float32),
    )(z, a, w1, b1.reshape(1, dout), w2, b2.reshape(1, dout), s, t)


def _final_body(z_ref, a_ref, w1_ref, b1_ref, w2_ref, b2_ref, s_ref, t_ref,
                batch_ref, f1w_ref, f1b_ref, f2w_ref, f2b_ref, f3w_ref, f3b_ref,
                o_ref, acc_ref):
    i = pl.program_id(0)

    @pl.when(i == 0)
    def _():
        acc_ref[...] = jnp.zeros_like(acc_ref)

    h = z_ref[...] + a_ref[...]
    u = jnp.maximum(jnp.dot(h, w1_ref[...], preferred_element_type=jnp.float32)
                    + b1_ref[...], 0.0)
    y = jnp.dot(u, w2_ref[...], preferred_element_type=jnp.float32) + b2_ref[...]
    z3 = jnp.maximum(y, 0.0) * s_ref[...] + t_ref[...]

    b = batch_ref[0, 0, :]
    onehot = (b[:, None] == jax.lax.broadcasted_iota(jnp.int32, (_RB, _G), 1)
              ).astype(jnp.float32)
    acc_ref[...] += jax.lax.dot_general(
        onehot, z3, (((0,), (0,)), ((), ())), preferred_element_type=jnp.float32)

    @pl.when(i == _NB - 1)
    def _():
        p = acc_ref[...]
        h1 = jnp.maximum(jnp.dot(p, f1w_ref[...], preferred_element_type=jnp.float32)
                         + f1b_ref[...], 0.0)
        h2 = jnp.maximum(jnp.dot(h1, f2w_ref[...], preferred_element_type=jnp.float32)
                         + f2b_ref[...], 0.0)
        o_ref[...] = jnp.dot(h2, f3w_ref[...], preferred_element_type=jnp.float32) \
            + f3b_ref[...]


def _tc_final(z, a, w1, b1, w2, b2, g, bb, batch, f1w, f1b, f2w, f2b, f3w, f3b):
    din, dout = w1.shape
    s = (g * _BN_SCALE).reshape(1, dout)
    t = bb.reshape(1, dout)
    batch3 = batch.reshape(_NB, 1, _RB)
    return pl.pallas_call(
        _final_body,
        grid=(_NB,),
        in_specs=[
            pl.BlockSpec((_RB, din), lambda i: (i, 0)),
            pl.BlockSpec((_RB, din), lambda i: (i, 0)),
            pl.BlockSpec((din, dout), lambda i: (0, 0)),
            pl.BlockSpec((1, dout), lambda i: (0, 0)),
            pl.BlockSpec((dout, dout), lambda i: (0, 0)),
            pl.BlockSpec((1, dout), lambda i: (0, 0)),
            pl.BlockSpec((1, dout), lambda i: (0, 0)),
            pl.BlockSpec((1, dout), lambda i: (0, 0)),
            pl.BlockSpec((1, 1, _RB), lambda i: (i, 0, 0)),
            pl.BlockSpec((dout, 128), lambda i: (0, 0)),
            pl.BlockSpec((1, 128), lambda i: (0, 0)),
            pl.BlockSpec((128, _G), lambda i: (0, 0)),
            pl.BlockSpec((1, _G), lambda i: (0, 0)),
            pl.BlockSpec((_G, _C), lambda i: (0, 0)),
            pl.BlockSpec((1, _C), lambda i: (0, 0)),
        ],
        out_specs=pl.BlockSpec((_G, _C), lambda i: (0, 0)),
        out_shape=jax.ShapeDtypeStruct((_G, _C), jnp.float32),
        scratch_shapes=[pltpu.VMEM((_G, dout), jnp.float32)],
    )(z, a, w1, b1.reshape(1, dout), w2, b2.reshape(1, dout), s, t, batch3,
      f1w, f1b.reshape(1, -1), f2w, f2b.reshape(1, -1), f3w, f3b.reshape(1, -1))


def _aggregate(z, src, dst, ea, lw, lb):
    m = jax.nn.relu(z[src] + (ea * lw.reshape(1, -1) + lb))
    return jax.ops.segment_sum(m, dst, num_segments=_N)


def kernel(x, edge_index, edge_attr, batch, params):
    p = params
    src, dst = edge_index[0], edge_index[1]
    ea = edge_attr.reshape(-1, 1)

    a1 = _aggregate(x, src, dst, ea, p['lin_e1_w'], p['lin_e1_b'])
    z1 = _tc_layer(x, a1, p['g1_w1'], p['g1_b1'], p['g1_w2'], p['g1_b2'],
                   p['bn1_g'], p['bn1_b'])
    a2 = _aggregate(z1, src, dst, ea, p['lin_e2_w'], p['lin_e2_b'])
    z2 = _tc_layer(z1, a2, p['g2_w1'], p['g2_b1'], p['g2_w2'], p['g2_b2'],
                   p['bn2_g'], p['bn2_b'])
    a3 = _aggregate(z2, src, dst, ea, p['lin_e3_w'], p['lin_e3_b'])
    out = _tc_final(z2, a3, p['g3_w1'], p['g3_b1'], p['g3_w2'], p['g3_b2'],
                    p['bn3_g'], p['bn3_b'], batch,
                    p['fc1_w'], p['fc1_b'], p['fc2_w'], p['fc2_b'],
                    p['fc3_w'], p['fc3_b'])
    return out


# speedup vs baseline: 1.4778x; 1.4304x over previous
import functools

import jax
import jax.numpy as jnp
from jax import lax
from jax.experimental import pallas as pl
from jax.experimental.pallas import tpu as pltpu
from jax.experimental.pallas import tpu_sc as plsc

_N = 10000
_E = 320000
_G = 64
_C = 10
_RB = 1000
_NB = _N // _RB

_NTILES = 32
_BLK = 128
_NBLK = 80
_EPT = _NBLK * _BLK
_ACC_N = 10112
_RPS = _ACC_N // 16
_CB = 16

_BN_SCALE = 1.0 / (1.0 + 1e-5) ** 0.5


def _make_sc_layer(d):
    nchunk = d // 16
    mesh = plsc.VectorSubcoreMesh(core_axis_name="c", subcore_axis_name="s")

    @functools.partial(
        pl.kernel,
        out_type=jax.ShapeDtypeStruct((2, _ACC_N, d), jnp.float32),
        mesh=mesh,
        scratch_types=[
            pltpu.VMEM_SHARED((_ACC_N, d), jnp.float32),
            pltpu.VMEM((_CB, _BLK), jnp.int32),
            pltpu.VMEM((_CB, _BLK), jnp.int32),
            pltpu.VMEM((_CB, _BLK), jnp.float32),
            pltpu.VMEM((_BLK, d), jnp.float32),
            pltpu.VMEM((_BLK, d), jnp.float32),
            pltpu.VMEM((2, d), jnp.float32),
            pltpu.SemaphoreType.DMA,
        ],
    )
    def sc_layer(z_hbm, src_hbm, dst_hbm, ea_hbm, lwb_hbm, zeros_hbm, out_hbm,
                 acc, src_c, dst_c, ea_c, rows, msg, lwb, gsem):
        cid = lax.axis_index("c")
        sid = lax.axis_index("s")
        w = cid * 16 + sid

        pltpu.sync_copy(lwb_hbm, lwb)
        pltpu.sync_copy(zeros_hbm, acc.at[pl.ds(sid * _RPS, _RPS)])
        plsc.subcore_barrier()

        def chunk(ch, carry0):
            pltpu.sync_copy(src_hbm.at[w, pl.ds(ch * _CB, _CB)], src_c)
            pltpu.sync_copy(dst_hbm.at[w, pl.ds(ch * _CB, _CB)], dst_c)
            pltpu.sync_copy(ea_hbm.at[w, pl.ds(ch * _CB, _CB)], ea_c)

            def block(b, carry):
                pltpu.async_copy(z_hbm.at[src_c.at[b]], rows, gsem).wait()

                def row_group(rr, c2):
                    eav = ea_c[b, pl.ds(rr * 16, 16)]
                    for r16 in range(16):
                        r = rr * 16 + r16
                        eab = eav[r16]
                        for c in range(nchunk):
                            sl = pl.ds(c * 16, 16)
                            msg[r, sl] = jnp.maximum(
                                rows[r, sl] + eab * lwb[0, sl] + lwb[1, sl],
                                0.0)
                    return c2
                lax.fori_loop(0, _BLK // 16, row_group, 0)
                pltpu.sync_copy(msg, acc.at[dst_c.at[b]], add=True)
                return carry
            lax.fori_loop(0, _CB, block, 0)
            return carry0
        lax.fori_loop(0, _NBLK // _CB, chunk, 0)
        plsc.subcore_barrier()
        pltpu.sync_copy(acc.at[pl.ds(sid * _RPS, _RPS)],
                        out_hbm.at[cid, pl.ds(sid * _RPS, _RPS)])

    return sc_layer


_sc_layer_128 = _make_sc_layer(128)


def _aggregate(z, srcp, dstp, eap, lw, lb, zeros):
    d = lw.size
    lwb = jnp.stack([lw.reshape(d), lb])
    if d < 128:
        lwb = jnp.pad(lwb, ((0, 0), (0, 128 - d)))
    return _sc_layer_128(z, srcp, dstp, eap, lwb, zeros)


def _layer_body(z_ref, a_ref, w1_ref, b1_ref, w2_ref, b2_ref, s_ref, t_ref,
                o_ref):
    h = z_ref[...] + a_ref[0] + a_ref[1]
    u = jnp.maximum(jnp.dot(h, w1_ref[...], preferred_element_type=jnp.float32)
                    + b1_ref[...], 0.0)
    y = jnp.dot(u, w2_ref[...], preferred_element_type=jnp.float32) + b2_ref[...]
    o_ref[...] = jnp.maximum(y, 0.0) * s_ref[...] + t_ref[...]


def _tc_layer(z, a, w1, b1, w2, b2, g, bb, dout_pad=None):
    din, dmid = w1.shape
    dout = w2.shape[1]
    s = (g * _BN_SCALE).reshape(1, dout)
    t = bb.reshape(1, dout)
    w2p, b2p = w2, b2.reshape(1, dout)
    if dout_pad is not None and dout_pad > dout:
        pad = dout_pad - dout
        w2p = jnp.pad(w2, ((0, 0), (0, pad)))
        b2p = jnp.pad(b2p, ((0, 0), (0, pad)))
        s = jnp.pad(s, ((0, 0), (0, pad)))
        t = jnp.pad(t, ((0, 0), (0, pad)))
        dout = dout_pad
    return pl.pallas_call(
        _layer_body,
        grid=(_NB,),
        in_specs=[
            pl.BlockSpec((_RB, din), lambda i: (i, 0)),
            pl.BlockSpec((2, _RB, din), lambda i: (0, i, 0)),
            pl.BlockSpec((din, dmid), lambda i: (0, 0)),
            pl.BlockSpec((1, dmid), lambda i: (0, 0)),
            pl.BlockSpec((dmid, dout), lambda i: (0, 0)),
            pl.BlockSpec((1, dout), lambda i: (0, 0)),
            pl.BlockSpec((1, dout), lambda i: (0, 0)),
            pl.BlockSpec((1, dout), lambda i: (0, 0)),
        ],
        out_specs=pl.BlockSpec((_RB, dout), lambda i: (i, 0)),
        out_shape=jax.ShapeDtypeStruct((_N, dout), jnp.float32),
    )(z, a, w1, b1.reshape(1, dmid), w2p, b2p, s, t)


def _final_body(z_ref, a_ref, w1_ref, b1_ref, w2_ref, b2_ref, s_ref, t_ref,
                batch_ref, f1w_ref, f1b_ref, f2w_ref, f2b_ref, f3w_ref,
                f3b_ref, o_ref, acc_ref):
    i = pl.program_id(0)

    @pl.when(i == 0)
    def _():
        acc_ref[...] = jnp.zeros_like(acc_ref)

    h = z_ref[...] + a_ref[0] + a_ref[1]
    u = jnp.maximum(jnp.dot(h, w1_ref[...], preferred_element_type=jnp.float32)
                    + b1_ref[...], 0.0)
    y = jnp.dot(u, w2_ref[...], preferred_element_type=jnp.float32) + b2_ref[...]
    z3 = jnp.maximum(y, 0.0) * s_ref[...] + t_ref[...]

    b = batch_ref[0, 0, :]
    onehot = (b[:, None] == jax.lax.broadcasted_iota(jnp.int32, (_RB, _G), 1)
              ).astype(jnp.float32)
    acc_ref[...] += jax.lax.dot_general(
        onehot, z3, (((0,), (0,)), ((), ())), preferred_element_type=jnp.float32)

    @pl.when(i == _NB - 1)
    def _():
        p = acc_ref[...]
        h1 = jnp.maximum(jnp.dot(p, f1w_ref[...],
                                 preferred_element_type=jnp.float32)
                         + f1b_ref[...], 0.0)
        h2 = jnp.maximum(jnp.dot(h1, f2w_ref[...],
                                 preferred_element_type=jnp.float32)
                         + f2b_ref[...], 0.0)
        o_ref[...] = jnp.dot(h2, f3w_ref[...],
                             preferred_element_type=jnp.float32) + f3b_ref[...]


def _tc_final(z, a, w1, b1, w2, b2, g, bb, batch, f1w, f1b, f2w, f2b, f3w,
              f3b):
    din, dout = w1.shape
    s = (g * _BN_SCALE).reshape(1, dout)
    t = bb.reshape(1, dout)
    batch3 = batch.reshape(_NB, 1, _RB)
    return pl.pallas_call(
        _final_body,
        grid=(_NB,),
        in_specs=[
            pl.BlockSpec((_RB, din), lambda i: (i, 0)),
            pl.BlockSpec((2, _RB, din), lambda i: (0, i, 0)),
            pl.BlockSpec((din, dout), lambda i: (0, 0)),
            pl.BlockSpec((1, dout), lambda i: (0, 0)),
            pl.BlockSpec((dout, dout), lambda i: (0, 0)),
            pl.BlockSpec((1, dout), lambda i: (0, 0)),
            pl.BlockSpec((1, dout), lambda i: (0, 0)),
            pl.BlockSpec((1, dout), lambda i: (0, 0)),
            pl.BlockSpec((1, 1, _RB), lambda i: (i, 0, 0)),
            pl.BlockSpec((dout, 128), lambda i: (0, 0)),
            pl.BlockSpec((1, 128), lambda i: (0, 0)),
            pl.BlockSpec((128, _G), lambda i: (0, 0)),
            pl.BlockSpec((1, _G), lambda i: (0, 0)),
            pl.BlockSpec((_G, _C), lambda i: (0, 0)),
            pl.BlockSpec((1, _C), lambda i: (0, 0)),
        ],
        out_specs=pl.BlockSpec((_G, _C), lambda i: (0, 0)),
        out_shape=jax.ShapeDtypeStruct((_G, _C), jnp.float32),
        scratch_shapes=[pltpu.VMEM((_G, dout), jnp.float32)],
    )(z, a, w1, b1.reshape(1, dout), w2, b2.reshape(1, dout), s, t, batch3,
      f1w, f1b.reshape(1, -1), f2w, f2b.reshape(1, -1), f3w, f3b.reshape(1, -1))


def kernel(x, edge_index, edge_attr, batch, params):
    p = params
    src, dst = edge_index[0], edge_index[1]
    epg = _E // _NTILES
    pad = _EPT - epg
    srcp = jnp.pad(src.reshape(_NTILES, epg),
                   ((0, 0), (0, pad))).reshape(_NTILES, _NBLK, _BLK)
    dstp = jnp.pad(dst.reshape(_NTILES, epg), ((0, 0), (0, pad)),
                   constant_values=_N).reshape(_NTILES, _NBLK, _BLK)
    eap = jnp.pad(edge_attr.reshape(_NTILES, epg),
                  ((0, 0), (0, pad))).reshape(_NTILES, _NBLK, _BLK)
    z128 = jnp.zeros((_RPS, 128), jnp.float32)

    a1 = _aggregate(x, srcp, dstp, eap, p['lin_e1_w'], p['lin_e1_b'], z128)
    z1 = _tc_layer(x, a1, p['g1_w1'], p['g1_b1'], p['g1_w2'], p['g1_b2'],
                   p['bn1_g'], p['bn1_b'])
    a2 = _aggregate(z1, srcp, dstp, eap, p['lin_e2_w'], p['lin_e2_b'], z128)
    z2 = _tc_layer(z1, a2, p['g2_w1'], p['g2_b1'], p['g2_w2'], p['g2_b2'],
                   p['bn2_g'], p['bn2_b'], dout_pad=128)
    a3 = _aggregate(z2, srcp, dstp, eap, p['lin_e3_w'], p['lin_e3_b'], z128)
    w31 = jnp.pad(p['g3_w1'], ((0, 64), (0, 0)))
    out = _tc_final(z2, a3, w31, p['g3_b1'], p['g3_w2'], p['g3_b2'],
                    p['bn3_g'], p['bn3_b'], batch,
                    p['fc1_w'], p['fc1_b'], p['fc2_w'], p['fc2_b'],
                    p['fc3_w'], p['fc3_b'])
    return out
